# baseline (speedup 1.0000x reference)
import functools

import jax
import jax.numpy as jnp
from jax import lax
from jax.experimental import pallas as pl
from jax.experimental.pallas import tpu as pltpu
from jax.experimental.pallas import tpu_sc as plsc

_KE = 14.3996454784255
_A_EXP = 0.3
_A_PREFACTOR = 0.4543
_INV_APREF = 1.0 / (_A_PREFACTOR * 0.529)
_C0, _C1, _C2, _C3 = 0.1818, 0.5099, 0.2802, 0.02817
_covl = [0.2, 0.31, 0.28, 1.28, 0.96, 0.84, 0.76, 0.71, 0.66, 0.57, 0.58,
         1.66, 1.41, 1.21, 1.11, 1.07, 1.05, 1.02, 1.06, 2.03, 1.76, 1.70,
         1.60, 1.53, 1.39, 1.39, 1.32, 1.26, 1.24, 1.32, 1.22, 1.22, 1.20,
         1.19, 1.20, 1.20, 1.16, 2.20, 1.95, 1.90, 1.75, 1.64, 1.54, 1.47,
         1.46, 1.42, 1.39, 1.45, 1.44, 1.42, 1.39, 1.39, 1.38, 1.39, 1.40,
         2.44, 2.15, 2.07, 2.04, 2.03, 2.01, 1.99, 1.98, 1.98, 1.96, 1.94,
         1.92, 1.92, 1.89, 1.90, 1.87, 1.87, 1.75, 1.70, 1.62, 1.51, 1.44,
         1.41, 1.36, 1.36, 1.32, 1.45, 1.46, 1.48, 1.40, 1.50, 1.50, 2.60,
         2.21, 2.15, 2.06, 2.00, 1.96, 1.90, 1.87, 1.80, 1.69]
_covl = _covl + [0.2] * (128 - len(_covl))

_CHUNK_ROWS = 16
_CHUNK = _CHUNK_ROWS * 128
_NW = 32


def _class_table_body(nelem, attrs_ref, azr_ref, azc_ref, covr_ref, covc_ref,
                      c_ref, zz_ref, ia_ref, rm_ref):
    a = attrs_ref[...]
    m = jnp.max(a, axis=1, keepdims=True)
    io = lax.broadcasted_iota(jnp.int32, a.shape, 1)
    cls = jnp.min(jnp.where(a >= m, io, nelem), axis=1)
    c_ref[...] = cls.reshape(1, 1, -1)

    azr = azr_ref[...].astype(jnp.float32)
    azc = azc_ref[...].astype(jnp.float32)
    zz_ref[...] = azc * azr
    zpr = jnp.exp(_A_EXP * jnp.log(azr))
    zpc = jnp.exp(_A_EXP * jnp.log(azc))
    ia_ref[...] = (zpc + zpr) * _INV_APREF

    ohr = lax.broadcasted_iota(jnp.int32, (128, 16), 0) == azr_ref[...]
    crr = jnp.sum(jnp.where(ohr, covc_ref[...], 0.0), axis=0, keepdims=True)
    ohc = lax.broadcasted_iota(jnp.int32, (16, 128), 1) == azc_ref[...]
    crc = jnp.sum(jnp.where(ohc, covr_ref[...], 0.0), axis=1, keepdims=True)
    rm_ref[...] = crc + crr


def _sc_body(nch, x_hbm, ei_hbm, c_hbm, zz_hbm, ia_hbm, rm_hbm, z_hbm,
             out_hbm, c_vm, zz_vm, ia_vm, rm_vm,
             s0, s1, r0, r1, x0, x1, v0, v1,
             acc_sh, in_sem, sc_sem):
    cid = lax.axis_index("core")
    sid = lax.axis_index("subcore")
    wid = sid * 2 + cid
    bufs = ((s0, r0, x0, v0), (s1, r1, x1, v1))

    pltpu.sync_copy(c_hbm, c_vm)
    pltpu.sync_copy(zz_hbm, zz_vm)
    pltpu.sync_copy(ia_hbm, ia_vm)
    pltpu.sync_copy(rm_hbm, rm_vm)

    @pl.when(sid == 0)
    def _():
        pltpu.sync_copy(z_hbm, acc_sh)

    plsc.subcore_barrier()

    base, rem = nch // _NW, nch % _NW
    n_my = base + jnp.where(wid < rem, 1, 0)

    def issue_inputs(slot, ch):
        s_b, r_b, x_b, _ = bufs[slot]
        pltpu.async_copy(ei_hbm.at[0, ch], s_b, in_sem.at[slot])
        pltpu.async_copy(ei_hbm.at[1, ch], r_b, in_sem.at[slot])
        pltpu.async_copy(x_hbm.at[ch], x_b, in_sem.at[slot])

    def wait_inputs(slot, ch):
        s_b, r_b, x_b, _ = bufs[slot]
        pltpu.make_async_copy(ei_hbm.at[0, ch], s_b, in_sem.at[slot]).wait()
        pltpu.make_async_copy(ei_hbm.at[1, ch], r_b, in_sem.at[slot]).wait()
        pltpu.make_async_copy(x_hbm.at[ch], x_b, in_sem.at[slot]).wait()

    def drain_scatter():
        pltpu.make_async_copy(x_hbm.at[0], v0, sc_sem).wait()

    def compute_chunk(slot):
        s_b, r_b, x_b, v_b = bufs[slot]

        @plsc.parallel_loop(0, _CHUNK_ROWS, 1, unroll=4)
        def _(j):
            for kk in range(0, 128, 16):
                s16 = s_b[j, pl.ds(kk, 16)]
                r16 = r_b[j, pl.ds(kk, 16)]
                xx = x_b[j, pl.ds(kk, 16)]
                cs = plsc.load_gather(c_vm, [s16])
                cr_ = plsc.load_gather(c_vm, [r16])
                p = cs * 16 + cr_
                zz = plsc.load_gather(zz_vm, [p])
                ia = plsc.load_gather(ia_vm, [p])
                rm = plsc.load_gather(rm_vm, [p])
                roa = xx * ia
                phi = (_C0 * jnp.exp(-3.2 * roa)
                       + _C1 * jnp.exp(-0.9423 * roa)
                       + _C2 * jnp.exp(-0.4028 * roa)
                       + _C3 * jnp.exp(-0.2016 * roa))
                rcp = 1.0 / (xx * rm)
                t = xx * xx * rcp
                xinv = rm * rcp
                t2 = t * t
                t6 = t2 * t2 * t2
                env = 1.0 - 28.0 * t6 + 48.0 * t6 * t - 21.0 * t6 * t2
                env = jnp.where(xx < rm, env, 0.0)
                v_b[j, pl.ds(kk, 16)] = (0.5 * _KE) * zz * xinv * phi * env

    issue_inputs(0, wid)
    nhalf = lax.div(n_my + 1, 2)

    @pl.loop(0, nhalf)
    def _(h):
        for b in range(2):
            i = h * 2 + b

            @pl.when(i < n_my)
            def _():
                s_b, r_b, x_b, v_b = bufs[b]
                ch = wid + i * _NW
                wait_inputs(b, ch)
                compute_chunk(b)

                @pl.when(i >= 1)
                def _():
                    drain_scatter()

                @pl.when(i + 1 < n_my)
                def _():
                    issue_inputs(1 - b, ch + _NW)

                @pl.loop(0, _CHUNK_ROWS)
                def _(j):
                    pltpu.async_copy(v_b.at[j], acc_sh.at[r_b.at[j]],
                                     sc_sem, add=True)

    drain_scatter()
    plsc.subcore_barrier()

    @pl.when(sid == 0)
    def _():
        pltpu.sync_copy(acc_sh, out_hbm.at[cid])


def _combine_body(p_ref, o_ref):
    o_ref[...] = p_ref[0] + p_ref[1]


def kernel(x, node_attrs, edge_index, atomic_numbers):
    n, nelem = node_attrs.shape
    e = x.shape[0]
    assert e % _CHUNK == 0 and n % 5000 == 0
    nch = e // _CHUNK
    nblk = n // 5000

    azr = jnp.ones((1, 16), jnp.int32).at[0, :nelem].set(atomic_numbers)
    azc = azr.reshape(16, 1)
    covr = jnp.array(_covl, dtype=jnp.float32).reshape(1, 128)
    covc = covr.reshape(128, 1)

    c3d, zz2d, ia2d, rm2d = pl.pallas_call(
        functools.partial(_class_table_body, nelem),
        grid=(nblk,),
        in_specs=[pl.BlockSpec((5000, nelem), lambda i: (i, 0)),
                  pl.BlockSpec((1, 16), lambda i: (0, 0)),
                  pl.BlockSpec((16, 1), lambda i: (0, 0)),
                  pl.BlockSpec((1, 128), lambda i: (0, 0)),
                  pl.BlockSpec((128, 1), lambda i: (0, 0))],
        out_specs=[pl.BlockSpec((1, 1, 5000), lambda i: (i, 0, 0)),
                   pl.BlockSpec((16, 16), lambda i: (0, 0)),
                   pl.BlockSpec((16, 16), lambda i: (0, 0)),
                   pl.BlockSpec((16, 16), lambda i: (0, 0))],
        out_shape=[jax.ShapeDtypeStruct((nblk, 1, 5000), jnp.int32),
                   jax.ShapeDtypeStruct((16, 16), jnp.float32),
                   jax.ShapeDtypeStruct((16, 16), jnp.float32),
                   jax.ShapeDtypeStruct((16, 16), jnp.float32)],
    )(node_attrs, azr, azc, covr, covc)

    c1d = c3d.reshape(n)
    x4 = x.reshape(nch, _CHUNK_ROWS, 128)
    ei4 = edge_index.reshape(2, nch, _CHUNK_ROWS, 128)
    zeros_n = jnp.zeros((n,), jnp.float32)

    sc_kernel = pl.kernel(
        functools.partial(_sc_body, nch),
        out_type=jax.ShapeDtypeStruct((2, n), jnp.float32),
        mesh=plsc.VectorSubcoreMesh(
            core_axis_name="core", subcore_axis_name="subcore",
            num_cores=2, num_subcores=16),
        compiler_params=pltpu.CompilerParams(needs_layout_passes=False),
        scratch_types=[
            pltpu.VMEM((n,), jnp.int32),
            pltpu.VMEM((256,), jnp.float32),
            pltpu.VMEM((256,), jnp.float32),
            pltpu.VMEM((256,), jnp.float32),
            pltpu.VMEM((_CHUNK_ROWS, 128), jnp.int32),
            pltpu.VMEM((_CHUNK_ROWS, 128), jnp.int32),
            pltpu.VMEM((_CHUNK_ROWS, 128), jnp.int32),
            pltpu.VMEM((_CHUNK_ROWS, 128), jnp.int32),
            pltpu.VMEM((_CHUNK_ROWS, 128), jnp.float32),
            pltpu.VMEM((_CHUNK_ROWS, 128), jnp.float32),
            pltpu.VMEM((_CHUNK_ROWS, 128), jnp.float32),
            pltpu.VMEM((_CHUNK_ROWS, 128), jnp.float32),
            pltpu.VMEM_SHARED((n,), jnp.float32),
            pltpu.SemaphoreType.DMA((2,)),
            pltpu.SemaphoreType.DMA,
        ],
    )
    partial = sc_kernel(x4, ei4, c1d, zz2d.reshape(256), ia2d.reshape(256),
                        rm2d.reshape(256), zeros_n)

    p3 = partial.reshape(2, nblk, 5000)
    out2 = pl.pallas_call(
        _combine_body,
        out_shape=jax.ShapeDtypeStruct((nblk, 5000), jnp.float32),
    )(p3)
    return out2.reshape(n)


# speedup vs baseline: 1.7205x; 1.7205x over previous
import functools

import jax
import jax.numpy as jnp
from jax import lax
from jax.experimental import pallas as pl
from jax.experimental.pallas import tpu as pltpu
from jax.experimental.pallas import tpu_sc as plsc

_KE = 14.3996454784255
_A_EXP = 0.3
_A_PREFACTOR = 0.4543
_INV_APREF = 1.0 / (_A_PREFACTOR * 0.529)
_C0, _C1, _C2, _C3 = 0.1818, 0.5099, 0.2802, 0.02817
_covl = [0.2, 0.31, 0.28, 1.28, 0.96, 0.84, 0.76, 0.71, 0.66, 0.57, 0.58,
         1.66, 1.41, 1.21, 1.11, 1.07, 1.05, 1.02, 1.06, 2.03, 1.76, 1.70,
         1.60, 1.53, 1.39, 1.39, 1.32, 1.26, 1.24, 1.32, 1.22, 1.22, 1.20,
         1.19, 1.20, 1.20, 1.16, 2.20, 1.95, 1.90, 1.75, 1.64, 1.54, 1.47,
         1.46, 1.42, 1.39, 1.45, 1.44, 1.42, 1.39, 1.39, 1.38, 1.39, 1.40,
         2.44, 2.15, 2.07, 2.04, 2.03, 2.01, 1.99, 1.98, 1.98, 1.96, 1.94,
         1.92, 1.92, 1.89, 1.90, 1.87, 1.87, 1.75, 1.70, 1.62, 1.51, 1.44,
         1.41, 1.36, 1.36, 1.32, 1.45, 1.46, 1.48, 1.40, 1.50, 1.50, 2.60,
         2.21, 2.15, 2.06, 2.00, 1.96, 1.90, 1.87, 1.80, 1.69]
_covl = _covl + [0.2] * (128 - len(_covl))

_CHUNK_ROWS = 16
_CHUNK = _CHUNK_ROWS * 128
_NW = 32


def _class_table_body(nelem, attrs_ref, azr_ref, azc_ref, covr_ref, covc_ref,
                      c_ref, zz_ref, ia_ref, rm_ref):
    a = attrs_ref[...]
    m = jnp.max(a, axis=1, keepdims=True)
    io = lax.broadcasted_iota(jnp.int32, a.shape, 1)
    cls = jnp.min(jnp.where(a >= m, io, nelem), axis=1)
    c_ref[...] = cls.reshape(1, 1, -1)

    azr = azr_ref[...].astype(jnp.float32)
    azc = azc_ref[...].astype(jnp.float32)
    zz_ref[...] = azc * azr
    zpr = jnp.exp(_A_EXP * jnp.log(azr))
    zpc = jnp.exp(_A_EXP * jnp.log(azc))
    ia_ref[...] = (zpc + zpr) * _INV_APREF

    ohr = lax.broadcasted_iota(jnp.int32, (128, 16), 0) == azr_ref[...]
    crr = jnp.sum(jnp.where(ohr, covc_ref[...], 0.0), axis=0, keepdims=True)
    ohc = lax.broadcasted_iota(jnp.int32, (16, 128), 1) == azc_ref[...]
    crc = jnp.sum(jnp.where(ohc, covr_ref[...], 0.0), axis=1, keepdims=True)
    rm_ref[...] = crc + crr


def _sc_body(nch, x_hbm, ei_hbm, c_hbm, zz_hbm, ia_hbm, rm_hbm, z_hbm,
             out_hbm, c_vm, zz_vm, ia_vm, rm_vm,
             s0, s1, r0, r1, x0, x1, v0, v1,
             acc_sh, in_sem, sc_sem):
    cid = lax.axis_index("core")
    sid = lax.axis_index("subcore")
    wid = sid * 2 + cid
    bufs = ((s0, r0, x0, v0), (s1, r1, x1, v1))

    pltpu.sync_copy(c_hbm, c_vm)
    pltpu.sync_copy(zz_hbm, zz_vm)
    pltpu.sync_copy(ia_hbm, ia_vm)
    pltpu.sync_copy(rm_hbm, rm_vm)

    @pl.when(sid == 0)
    def _():
        pltpu.sync_copy(z_hbm, acc_sh)

    plsc.subcore_barrier()

    base, rem = nch // _NW, nch % _NW
    n_my = base + jnp.where(wid < rem, 1, 0)

    def issue_inputs(slot, ch):
        s_b, r_b, x_b, _ = bufs[slot]
        pltpu.async_copy(ei_hbm.at[0, ch], s_b, in_sem.at[slot])
        pltpu.async_copy(ei_hbm.at[1, ch], r_b, in_sem.at[slot])
        pltpu.async_copy(x_hbm.at[ch], x_b, in_sem.at[slot])

    def wait_inputs(slot, ch):
        s_b, r_b, x_b, _ = bufs[slot]
        pltpu.make_async_copy(ei_hbm.at[0, ch], s_b, in_sem.at[slot]).wait()
        pltpu.make_async_copy(ei_hbm.at[1, ch], r_b, in_sem.at[slot]).wait()
        pltpu.make_async_copy(x_hbm.at[ch], x_b, in_sem.at[slot]).wait()

    def drain_scatter():
        pltpu.make_async_copy(x_hbm.at[0], v0, sc_sem).wait()

    def compute_chunk(slot):
        s_b, r_b, x_b, v_b = bufs[slot]

        @plsc.parallel_loop(0, _CHUNK_ROWS, 1, unroll=1)
        def _(j):
            for kk in range(0, 128, 16):
                s16 = s_b[j, pl.ds(kk, 16)]
                r16 = r_b[j, pl.ds(kk, 16)]
                xx = x_b[j, pl.ds(kk, 16)]
                cs = plsc.load_gather(c_vm, [s16])
                cr_ = plsc.load_gather(c_vm, [r16])
                p = cs * 16 + cr_
                zz = plsc.load_gather(zz_vm, [p])
                ia = plsc.load_gather(ia_vm, [p])
                rm = plsc.load_gather(rm_vm, [p])
                roa = xx * ia
                phi = (_C0 * jnp.exp(-3.2 * roa)
                       + _C1 * jnp.exp(-0.9423 * roa)
                       + _C2 * jnp.exp(-0.4028 * roa)
                       + _C3 * jnp.exp(-0.2016 * roa))
                rcp = 1.0 / (xx * rm)
                t = xx * xx * rcp
                xinv = rm * rcp
                t2 = t * t
                t6 = t2 * t2 * t2
                env = 1.0 - 28.0 * t6 + 48.0 * t6 * t - 21.0 * t6 * t2
                env = jnp.where(xx < rm, env, 0.0)
                v_b[j, pl.ds(kk, 16)] = (0.5 * _KE) * zz * xinv * phi * env

    issue_inputs(0, wid)
    nhalf = lax.div(n_my + 1, 2)

    @pl.loop(0, nhalf)
    def _(h):
        for b in range(2):
            i = h * 2 + b

            @pl.when(i < n_my)
            def _():
                s_b, r_b, x_b, v_b = bufs[b]
                ch = wid + i * _NW
                wait_inputs(b, ch)
                compute_chunk(b)

                @pl.when(i >= 1)
                def _():
                    drain_scatter()

                @pl.when(i + 1 < n_my)
                def _():
                    issue_inputs(1 - b, ch + _NW)

                @pl.loop(0, _CHUNK_ROWS)
                def _(j):
                    pltpu.async_copy(v_b.at[j], acc_sh.at[r_b.at[j]],
                                     sc_sem, add=True)

    drain_scatter()
    plsc.subcore_barrier()

    @pl.when(sid == 0)
    def _():
        pltpu.sync_copy(acc_sh, out_hbm.at[cid])


def _combine_body(p_ref, o_ref):
    o_ref[...] = p_ref[0] + p_ref[1]


def kernel(x, node_attrs, edge_index, atomic_numbers):
    n, nelem = node_attrs.shape
    e = x.shape[0]
    assert e % _CHUNK == 0 and n % 5000 == 0
    nch = e // _CHUNK
    nblk = n // 5000

    azr = jnp.ones((1, 16), jnp.int32).at[0, :nelem].set(atomic_numbers)
    azc = azr.reshape(16, 1)
    covr = jnp.array(_covl, dtype=jnp.float32).reshape(1, 128)
    covc = covr.reshape(128, 1)

    c3d, zz2d, ia2d, rm2d = pl.pallas_call(
        functools.partial(_class_table_body, nelem),
        grid=(nblk,),
        in_specs=[pl.BlockSpec((5000, nelem), lambda i: (i, 0)),
                  pl.BlockSpec((1, 16), lambda i: (0, 0)),
                  pl.BlockSpec((16, 1), lambda i: (0, 0)),
                  pl.BlockSpec((1, 128), lambda i: (0, 0)),
                  pl.BlockSpec((128, 1), lambda i: (0, 0))],
        out_specs=[pl.BlockSpec((1, 1, 5000), lambda i: (i, 0, 0)),
                   pl.BlockSpec((16, 16), lambda i: (0, 0)),
                   pl.BlockSpec((16, 16), lambda i: (0, 0)),
                   pl.BlockSpec((16, 16), lambda i: (0, 0))],
        out_shape=[jax.ShapeDtypeStruct((nblk, 1, 5000), jnp.int32),
                   jax.ShapeDtypeStruct((16, 16), jnp.float32),
                   jax.ShapeDtypeStruct((16, 16), jnp.float32),
                   jax.ShapeDtypeStruct((16, 16), jnp.float32)],
    )(node_attrs, azr, azc, covr, covc)

    c1d = c3d.reshape(n)
    x4 = x.reshape(nch, _CHUNK_ROWS, 128)
    ei4 = edge_index.reshape(2, nch, _CHUNK_ROWS, 128)
    zeros_n = jnp.zeros((n,), jnp.float32)

    sc_kernel = pl.kernel(
        functools.partial(_sc_body, nch),
        out_type=jax.ShapeDtypeStruct((2, n), jnp.float32),
        mesh=plsc.VectorSubcoreMesh(
            core_axis_name="core", subcore_axis_name="subcore",
            num_cores=2, num_subcores=16),
        compiler_params=pltpu.CompilerParams(needs_layout_passes=False),
        scratch_types=[
            pltpu.VMEM((n,), jnp.int32),
            pltpu.VMEM((256,), jnp.float32),
            pltpu.VMEM((256,), jnp.float32),
            pltpu.VMEM((256,), jnp.float32),
            pltpu.VMEM((_CHUNK_ROWS, 128), jnp.int32),
            pltpu.VMEM((_CHUNK_ROWS, 128), jnp.int32),
            pltpu.VMEM((_CHUNK_ROWS, 128), jnp.int32),
            pltpu.VMEM((_CHUNK_ROWS, 128), jnp.int32),
            pltpu.VMEM((_CHUNK_ROWS, 128), jnp.float32),
            pltpu.VMEM((_CHUNK_ROWS, 128), jnp.float32),
            pltpu.VMEM((_CHUNK_ROWS, 128), jnp.float32),
            pltpu.VMEM((_CHUNK_ROWS, 128), jnp.float32),
            pltpu.VMEM_SHARED((n,), jnp.float32),
            pltpu.SemaphoreType.DMA((2,)),
            pltpu.SemaphoreType.DMA,
        ],
    )
    partial = sc_kernel(x4, ei4, c1d, zz2d.reshape(256), ia2d.reshape(256),
                        rm2d.reshape(256), zeros_n)

    p3 = partial.reshape(2, nblk, 5000)
    out2 = pl.pallas_call(
        _combine_body,
        out_shape=jax.ShapeDtypeStruct((nblk, 5000), jnp.float32),
    )(p3)
    return out2.reshape(n)
